# baseline (speedup 1.0000x reference)
import functools

import jax
import jax.numpy as jnp
from jax import lax
from jax.experimental import pallas as pl
from jax.experimental.pallas import tpu as pltpu
from jax.experimental.pallas import tpu_sc as plsc

_NC = 2
_NS = 16
_NT = _NC * _NS
_CH = 128
_LANES = 16


def _sc_mesh():
    return plsc.VectorSubcoreMesh(core_axis_name="c", subcore_axis_name="s")


def _sc_pass(src, dst, u, fill, n_pad, cpt, gather):
    rpt = n_pad // _NS
    nb = 8
    assert cpt % nb == 0
    scratch = [
        pltpu.VMEM((cpt, _CH), jnp.int32),
        pltpu.VMEM((nb, _CH, _LANES), jnp.float32),
        pltpu.VMEM_SHARED((n_pad, _LANES), jnp.float32),
        pltpu.SemaphoreType.DMA((nb,)),
        pltpu.SemaphoreType.DMA((nb,)),
    ]
    if gather:
        scratch.insert(0, pltpu.VMEM((cpt, _CH), jnp.int32))
        scratch.insert(3, pltpu.VMEM_SHARED((n_pad, _LANES), jnp.float32))

    def body(*refs):
        if gather:
            (src_hbm, dst_hbm, u_hbm, fill_hbm, out_hbm,
             src_v, dst_v, rows_v, u_sh, acc_sh, gsem, ssem) = refs
        else:
            (dst_hbm, fill_hbm, out_hbm, dst_v, rows_v, acc_sh,
             gsem, ssem) = refs
        c = lax.axis_index("c")
        s = lax.axis_index("s")
        wid = c * _NS + s

        pltpu.sync_copy(fill_hbm.at[0], rows_v.at[0])

        @pl.loop(0, rpt // _CH)
        def _(i):
            pltpu.sync_copy(rows_v.at[0],
                            acc_sh.at[pl.ds(s * rpt + i * _CH, _CH)])

        pltpu.sync_copy(dst_hbm.at[pl.ds(wid * cpt, cpt)], dst_v)
        if gather:
            pltpu.sync_copy(src_hbm.at[pl.ds(wid * cpt, cpt)], src_v)
            pltpu.sync_copy(u_hbm.at[pl.ds(s * rpt, rpt)],
                            u_sh.at[pl.ds(s * rpt, rpt)])
        else:
            for b in range(nb):
                pltpu.sync_copy(fill_hbm.at[1], rows_v.at[b])
        plsc.subcore_barrier()

        if gather:
            for b in range(nb):
                pltpu.async_copy(u_sh.at[src_v.at[b]], rows_v.at[b],
                                 gsem.at[b])

            @pl.loop(0, cpt // nb)
            def _(t):
                j0 = t * nb
                for b in range(nb):
                    pltpu.make_async_copy(u_sh.at[src_v.at[j0 + b]],
                                          rows_v.at[b], gsem.at[b]).wait()
                    pltpu.async_copy(rows_v.at[b],
                                     acc_sh.at[dst_v.at[j0 + b]],
                                     ssem.at[b], add=True)
                for b in range(nb):
                    jn = jnp.minimum(j0 + b + nb, cpt - 1)
                    pltpu.make_async_copy(rows_v.at[b],
                                          acc_sh.at[dst_v.at[j0 + b]],
                                          ssem.at[b]).wait()
                    pltpu.async_copy(u_sh.at[src_v.at[jn]], rows_v.at[b],
                                     gsem.at[b])

            for b in range(nb):
                pltpu.make_async_copy(u_sh.at[src_v.at[0]], rows_v.at[b],
                                      gsem.at[b]).wait()
        else:
            for b in range(nb):
                pltpu.async_copy(rows_v.at[b], acc_sh.at[dst_v.at[b]],
                                 ssem.at[b], add=True)

            @pl.loop(1, cpt // nb)
            def _(t):
                for b in range(nb):
                    pltpu.make_async_copy(rows_v.at[b],
                                          acc_sh.at[dst_v.at[b]],
                                          ssem.at[b]).wait()
                    pltpu.async_copy(rows_v.at[b],
                                     acc_sh.at[dst_v.at[t * nb + b]],
                                     ssem.at[b], add=True)

            for b in range(nb):
                pltpu.make_async_copy(rows_v.at[b], acc_sh.at[dst_v.at[0]],
                                      ssem.at[b]).wait()

        plsc.subcore_barrier()
        pltpu.sync_copy(acc_sh.at[pl.ds(s * rpt, rpt)],
                        out_hbm.at[c, pl.ds(s * rpt, rpt)])

    k = pl.kernel(
        body,
        out_type=jax.ShapeDtypeStruct((_NC, n_pad, _LANES), jnp.float32),
        mesh=_sc_mesh(),
        scratch_types=scratch,
        compiler_params=pltpu.CompilerParams(use_tc_tiling_on_sc=False),
    )
    if gather:
        return k(src, dst, u, fill)
    return k(dst, fill)


def _tc_mm_body(x_ref, w_ref, h_ref):
    h_ref[...] = jnp.dot(x_ref[...], w_ref[...],
                         preferred_element_type=jnp.float32)


def _tc_a_body(h_ref, d0_ref, d1_ref, u_ref, dv_ref):
    deg = d0_ref[...] + d1_ref[...] + 1.0
    dinv = lax.rsqrt(deg)
    u_ref[...] = dinv * h_ref[...]
    dv_ref[...] = dinv


def _tc_b_body(s0_ref, s1_ref, u1_ref, dv_ref, b1_ref, u2_ref):
    dv = dv_ref[...]
    z = dv * (s0_ref[...] + s1_ref[...] + u1_ref[...]) + b1_ref[...][0][None, :]
    u2_ref[...] = dv * jnp.maximum(z, 0.0)


def _tc_c_body(s0_ref, s1_ref, u2_ref, dv_ref, w2_ref, b2_ref, o_ref):
    agg = dv_ref[...] * (s0_ref[...] + s1_ref[...] + u2_ref[...])
    y = jnp.dot(agg, w2_ref[...], preferred_element_type=jnp.float32)
    y = y + b2_ref[...][0][None, :]
    m = jnp.max(y, axis=1, keepdims=True)
    lse = jnp.log(jnp.sum(jnp.exp(y - m), axis=1, keepdims=True)) + m
    o_ref[...] = y - lse


def kernel(x, edge_index, W1, b1, W2, b2):
    n, din = x.shape
    e = edge_index.shape[1]
    h = W1.shape[1]
    dout = W2.shape[1]
    assert h == _LANES

    n_pad = -(-n // (_NS * _CH)) * (_NS * _CH)
    cpt = -(-(-(-e // (_NT * _CH))) // 8) * 8
    e_pad = _NT * cpt * _CH

    xp = jnp.pad(x, ((0, n_pad - n), (0, 0)))
    src = jnp.pad(edge_index[0], (0, e_pad - e)).reshape(_NT * cpt, _CH)
    dst = jnp.pad(edge_index[1], (0, e_pad - e),
                  constant_values=n_pad - 1).reshape(_NT * cpt, _CH)
    fill = jnp.stack([jnp.zeros((_CH, _LANES), jnp.float32),
                      jnp.ones((_CH, _LANES), jnp.float32)])

    br = 1024
    grid = (n_pad // br,)
    row_spec = pl.BlockSpec((br, _LANES), lambda i: (i, 0))

    degp = _sc_pass(None, dst, None, fill, n_pad, cpt, gather=False)

    hm = pl.pallas_call(
        _tc_mm_body,
        grid=grid,
        in_specs=[pl.BlockSpec((br, din), lambda i: (i, 0)),
                  pl.BlockSpec((din, h), lambda i: (0, 0))],
        out_specs=row_spec,
        out_shape=jax.ShapeDtypeStruct((n_pad, h), jnp.float32),
    )(xp, W1)

    u1, dv = pl.pallas_call(
        _tc_a_body,
        grid=grid,
        in_specs=[row_spec, row_spec, row_spec],
        out_specs=[row_spec, row_spec],
        out_shape=[jax.ShapeDtypeStruct((n_pad, h), jnp.float32),
                   jax.ShapeDtypeStruct((n_pad, h), jnp.float32)],
    )(hm, degp[0], degp[1])

    s1p = _sc_pass(src, dst, u1, fill, n_pad, cpt, gather=True)

    u2 = pl.pallas_call(
        _tc_b_body,
        grid=grid,
        in_specs=[row_spec, row_spec, row_spec, row_spec,
                  pl.BlockSpec((8, h), lambda i: (0, 0))],
        out_specs=row_spec,
        out_shape=jax.ShapeDtypeStruct((n_pad, h), jnp.float32),
    )(s1p[0], s1p[1], u1, dv, jnp.broadcast_to(b1, (8, h)))

    s2p = _sc_pass(src, dst, u2, fill, n_pad, cpt, gather=True)

    out = pl.pallas_call(
        _tc_c_body,
        grid=grid,
        in_specs=[row_spec, row_spec, row_spec, row_spec,
                  pl.BlockSpec((h, dout), lambda i: (0, 0)),
                  pl.BlockSpec((8, dout), lambda i: (0, 0))],
        out_specs=pl.BlockSpec((br, dout), lambda i: (i, 0)),
        out_shape=jax.ShapeDtypeStruct((n_pad, dout), jnp.float32),
    )(s2p[0], s2p[1], u2, dv, W2, jnp.broadcast_to(b2, (8, dout)))

    return out[:n]


# speedup vs baseline: 1.0314x; 1.0314x over previous
import functools

import jax
import jax.numpy as jnp
from jax import lax
from jax.experimental import pallas as pl
from jax.experimental.pallas import tpu as pltpu
from jax.experimental.pallas import tpu_sc as plsc

_NC = 2
_NS = 16
_NT = _NC * _NS
_CH = 128
_LANES = 16


def _sc_mesh():
    return plsc.VectorSubcoreMesh(core_axis_name="c", subcore_axis_name="s")


def _sc_pass(src, dst, u, fill, n_pad, cpt, gather):
    rpt = n_pad // _NS
    nb = 4
    assert cpt % nb == 0
    scratch = [
        pltpu.VMEM((cpt, _CH), jnp.int32),
        pltpu.VMEM((nb, _CH, _LANES), jnp.float32),
        pltpu.VMEM_SHARED((n_pad, _LANES), jnp.float32),
        pltpu.SemaphoreType.DMA((nb,)),
        pltpu.SemaphoreType.DMA((nb,)),
    ]
    if gather:
        scratch.insert(0, pltpu.VMEM((cpt, _CH), jnp.int32))
        scratch.insert(3, pltpu.VMEM_SHARED((n_pad, _LANES), jnp.float32))

    def body(*refs):
        if gather:
            (src_hbm, dst_hbm, u_hbm, fill_hbm, out_hbm,
             src_v, dst_v, rows_v, u_sh, acc_sh, gsem, ssem) = refs
        else:
            (dst_hbm, fill_hbm, out_hbm, dst_v, rows_v, acc_sh,
             gsem, ssem) = refs
        c = lax.axis_index("c")
        s = lax.axis_index("s")
        wid = c * _NS + s

        pltpu.sync_copy(fill_hbm.at[0], rows_v.at[0])

        @pl.loop(0, rpt // _CH)
        def _(i):
            pltpu.sync_copy(rows_v.at[0],
                            acc_sh.at[pl.ds(s * rpt + i * _CH, _CH)])

        pltpu.sync_copy(dst_hbm.at[pl.ds(wid * cpt, cpt)], dst_v)
        if gather:
            pltpu.sync_copy(src_hbm.at[pl.ds(wid * cpt, cpt)], src_v)
            pltpu.sync_copy(u_hbm.at[pl.ds(s * rpt, rpt)],
                            u_sh.at[pl.ds(s * rpt, rpt)])
        else:
            for b in range(nb):
                pltpu.sync_copy(fill_hbm.at[1], rows_v.at[b])
        plsc.subcore_barrier()

        if gather:
            for b in range(nb):
                pltpu.async_copy(u_sh.at[src_v.at[b]], rows_v.at[b],
                                 gsem.at[b])

            @pl.loop(0, cpt // nb)
            def _(t):
                j0 = t * nb
                for b in range(nb):
                    pltpu.make_async_copy(u_sh.at[src_v.at[j0 + b]],
                                          rows_v.at[b], gsem.at[b]).wait()
                    pltpu.async_copy(rows_v.at[b],
                                     acc_sh.at[dst_v.at[j0 + b]],
                                     ssem.at[b], add=True)
                for b in range(nb):
                    jn = jnp.minimum(j0 + b + nb, cpt - 1)
                    pltpu.make_async_copy(rows_v.at[b],
                                          acc_sh.at[dst_v.at[j0 + b]],
                                          ssem.at[b]).wait()
                    pltpu.async_copy(u_sh.at[src_v.at[jn]], rows_v.at[b],
                                     gsem.at[b])

            for b in range(nb):
                pltpu.make_async_copy(u_sh.at[src_v.at[0]], rows_v.at[b],
                                      gsem.at[b]).wait()
        else:
            for b in range(nb):
                pltpu.async_copy(rows_v.at[b], acc_sh.at[dst_v.at[b]],
                                 ssem.at[b], add=True)

            @pl.loop(1, cpt // nb)
            def _(t):
                for b in range(nb):
                    pltpu.make_async_copy(rows_v.at[b],
                                          acc_sh.at[dst_v.at[b]],
                                          ssem.at[b]).wait()
                    pltpu.async_copy(rows_v.at[b],
                                     acc_sh.at[dst_v.at[t * nb + b]],
                                     ssem.at[b], add=True)

            for b in range(nb):
                pltpu.make_async_copy(rows_v.at[b], acc_sh.at[dst_v.at[0]],
                                      ssem.at[b]).wait()

        plsc.subcore_barrier()
        pltpu.sync_copy(acc_sh.at[pl.ds(s * rpt, rpt)],
                        out_hbm.at[c, pl.ds(s * rpt, rpt)])

    k = pl.kernel(
        body,
        out_type=jax.ShapeDtypeStruct((_NC, n_pad, _LANES), jnp.float32),
        mesh=_sc_mesh(),
        scratch_types=scratch,
        compiler_params=pltpu.CompilerParams(use_tc_tiling_on_sc=False),
    )
    if gather:
        return k(src, dst, u, fill)
    return k(dst, fill)


def _tc_mm_body(x_ref, w_ref, h_ref):
    h_ref[...] = jnp.dot(x_ref[...], w_ref[...],
                         preferred_element_type=jnp.float32)


def _tc_a_body(h_ref, d0_ref, d1_ref, u_ref, dv_ref):
    deg = d0_ref[...] + d1_ref[...] + 1.0
    dinv = lax.rsqrt(deg)
    u_ref[...] = dinv * h_ref[...]
    dv_ref[...] = dinv


def _tc_b_body(s0_ref, s1_ref, u1_ref, dv_ref, b1_ref, u2_ref):
    dv = dv_ref[...]
    z = dv * (s0_ref[...] + s1_ref[...] + u1_ref[...]) + b1_ref[...][0][None, :]
    u2_ref[...] = dv * jnp.maximum(z, 0.0)


def _tc_c_body(s0_ref, s1_ref, u2_ref, dv_ref, w2_ref, b2_ref, o_ref):
    agg = dv_ref[...] * (s0_ref[...] + s1_ref[...] + u2_ref[...])
    y = jnp.dot(agg, w2_ref[...], preferred_element_type=jnp.float32)
    y = y + b2_ref[...][0][None, :]
    m = jnp.max(y, axis=1, keepdims=True)
    lse = jnp.log(jnp.sum(jnp.exp(y - m), axis=1, keepdims=True)) + m
    o_ref[...] = y - lse


def kernel(x, edge_index, W1, b1, W2, b2):
    n, din = x.shape
    e = edge_index.shape[1]
    h = W1.shape[1]
    dout = W2.shape[1]
    assert h == _LANES

    n_pad = -(-n // (_NS * _CH)) * (_NS * _CH)
    cpt = -(-(-(-e // (_NT * _CH))) // 16) * 16
    e_pad = _NT * cpt * _CH

    xp = jnp.pad(x, ((0, n_pad - n), (0, 0)))
    src = jnp.pad(edge_index[0], (0, e_pad - e)).reshape(_NT * cpt, _CH)
    dst = jnp.pad(edge_index[1], (0, e_pad - e),
                  constant_values=n_pad - 1).reshape(_NT * cpt, _CH)
    fill = jnp.stack([jnp.zeros((_CH, _LANES), jnp.float32),
                      jnp.ones((_CH, _LANES), jnp.float32)])

    br = 1024
    grid = (n_pad // br,)
    row_spec = pl.BlockSpec((br, _LANES), lambda i: (i, 0))

    degp = _sc_pass(None, dst, None, fill, n_pad, cpt, gather=False)

    hm = pl.pallas_call(
        _tc_mm_body,
        grid=grid,
        in_specs=[pl.BlockSpec((br, din), lambda i: (i, 0)),
                  pl.BlockSpec((din, h), lambda i: (0, 0))],
        out_specs=row_spec,
        out_shape=jax.ShapeDtypeStruct((n_pad, h), jnp.float32),
    )(xp, W1)

    u1, dv = pl.pallas_call(
        _tc_a_body,
        grid=grid,
        in_specs=[row_spec, row_spec, row_spec],
        out_specs=[row_spec, row_spec],
        out_shape=[jax.ShapeDtypeStruct((n_pad, h), jnp.float32),
                   jax.ShapeDtypeStruct((n_pad, h), jnp.float32)],
    )(hm, degp[0], degp[1])

    s1p = _sc_pass(src, dst, u1, fill, n_pad, cpt, gather=True)

    u2 = pl.pallas_call(
        _tc_b_body,
        grid=grid,
        in_specs=[row_spec, row_spec, row_spec, row_spec,
                  pl.BlockSpec((8, h), lambda i: (0, 0))],
        out_specs=row_spec,
        out_shape=jax.ShapeDtypeStruct((n_pad, h), jnp.float32),
    )(s1p[0], s1p[1], u1, dv, jnp.broadcast_to(b1, (8, h)))

    s2p = _sc_pass(src, dst, u2, fill, n_pad, cpt, gather=True)

    out = pl.pallas_call(
        _tc_c_body,
        grid=grid,
        in_specs=[row_spec, row_spec, row_spec, row_spec,
                  pl.BlockSpec((h, dout), lambda i: (0, 0)),
                  pl.BlockSpec((8, dout), lambda i: (0, 0))],
        out_specs=pl.BlockSpec((br, dout), lambda i: (i, 0)),
        out_shape=jax.ShapeDtypeStruct((n_pad, dout), jnp.float32),
    )(s2p[0], s2p[1], u2, dv, W2, jnp.broadcast_to(b2, (8, dout)))

    return out[:n]


# speedup vs baseline: 1.0980x; 1.0646x over previous
import functools

import jax
import jax.numpy as jnp
from jax import lax
from jax.experimental import pallas as pl
from jax.experimental.pallas import tpu as pltpu
from jax.experimental.pallas import tpu_sc as plsc

_NC = 2
_NS = 16
_NT = _NC * _NS
_CH = 128
_LANES = 16


def _sc_mesh():
    return plsc.VectorSubcoreMesh(core_axis_name="c", subcore_axis_name="s")


def _sc_pass(src, dst, u, fill, n_pad, cpt, gather):
    rpt = n_pad // _NS
    nb = 4
    assert cpt % nb == 0
    scratch = [
        pltpu.VMEM((cpt, _CH), jnp.int32),
        pltpu.VMEM((nb, _CH, _LANES), jnp.float32),
        pltpu.VMEM_SHARED((n_pad, _LANES), jnp.float32),
        pltpu.SemaphoreType.DMA((nb,)),
        pltpu.SemaphoreType.DMA((nb,)),
    ]
    if gather:
        scratch.insert(0, pltpu.VMEM((cpt, _CH), jnp.int32))
        scratch.insert(3, pltpu.VMEM_SHARED((n_pad, _LANES), jnp.float32))

    def body(*refs):
        if gather:
            (src_hbm, dst_hbm, u_hbm, fill_hbm, out_hbm,
             src_v, dst_v, rows_v, u_sh, acc_sh, gsem, ssem) = refs
        else:
            (dst_hbm, fill_hbm, out_hbm, dst_v, rows_v, acc_sh,
             gsem, ssem) = refs
        c = lax.axis_index("c")
        s = lax.axis_index("s")
        wid = c * _NS + s

        pltpu.sync_copy(fill_hbm.at[0], rows_v.at[0])

        @pl.loop(0, rpt // _CH)
        def _(i):
            pltpu.sync_copy(rows_v.at[0],
                            acc_sh.at[pl.ds(s * rpt + i * _CH, _CH)])

        pltpu.sync_copy(dst_hbm.at[pl.ds(wid * cpt, cpt)], dst_v)
        if gather:
            pltpu.sync_copy(src_hbm.at[pl.ds(wid * cpt, cpt)], src_v)
            pltpu.sync_copy(u_hbm.at[pl.ds(s * rpt, rpt)],
                            u_sh.at[pl.ds(s * rpt, rpt)])
        else:
            for b in range(nb):
                pltpu.sync_copy(fill_hbm.at[1], rows_v.at[b])
        plsc.subcore_barrier()

        if gather:
            for b in range(nb):
                pltpu.async_copy(u_sh.at[src_v.at[b]], rows_v.at[b],
                                 gsem.at[b])

            @pl.loop(0, cpt // nb)
            def _(t):
                j0 = t * nb
                for b in range(nb):
                    pltpu.make_async_copy(u_sh.at[src_v.at[j0 + b]],
                                          rows_v.at[b], gsem.at[b]).wait()
                    pltpu.async_copy(rows_v.at[b],
                                     acc_sh.at[dst_v.at[j0 + b]],
                                     ssem.at[b], add=True)
                for b in range(nb):
                    jn = jnp.minimum(j0 + b + nb, cpt - 1)
                    pltpu.make_async_copy(rows_v.at[b],
                                          acc_sh.at[dst_v.at[j0 + b]],
                                          ssem.at[b]).wait()
                    pltpu.async_copy(u_sh.at[src_v.at[jn]], rows_v.at[b],
                                     gsem.at[b])

            for b in range(nb):
                pltpu.make_async_copy(u_sh.at[src_v.at[0]], rows_v.at[b],
                                      gsem.at[b]).wait()
        else:
            for b in range(nb):
                pltpu.async_copy(rows_v.at[b], acc_sh.at[dst_v.at[b]],
                                 ssem.at[b], add=True)

            @pl.loop(1, cpt // nb)
            def _(t):
                for b in range(nb):
                    pltpu.make_async_copy(rows_v.at[b],
                                          acc_sh.at[dst_v.at[b]],
                                          ssem.at[b]).wait()
                    pltpu.async_copy(rows_v.at[b],
                                     acc_sh.at[dst_v.at[t * nb + b]],
                                     ssem.at[b], add=True)

            for b in range(nb):
                pltpu.make_async_copy(rows_v.at[b], acc_sh.at[dst_v.at[0]],
                                      ssem.at[b]).wait()

        plsc.subcore_barrier()
        pltpu.sync_copy(acc_sh.at[pl.ds(s * rpt, rpt)],
                        out_hbm.at[c, pl.ds(s * rpt, rpt)])

    k = pl.kernel(
        body,
        out_type=jax.ShapeDtypeStruct((_NC, n_pad, _LANES), jnp.float32),
        mesh=_sc_mesh(),
        scratch_types=scratch,
        compiler_params=pltpu.CompilerParams(use_tc_tiling_on_sc=False),
    )
    if gather:
        return k(src, dst, u, fill)
    return k(dst, fill)


def _tc_mm_body(x_ref, w_ref, h_ref):
    h_ref[...] = jnp.dot(x_ref[...], w_ref[...],
                         preferred_element_type=jnp.float32)


def _tc_a_body(h_ref, d0_ref, d1_ref, u_ref, dv_ref):
    deg = d0_ref[...] + d1_ref[...] + 1.0
    dinv = lax.rsqrt(deg)
    u_ref[...] = dinv * h_ref[...]
    dv_ref[...] = dinv


def _tc_b_body(s0_ref, s1_ref, u1_ref, dv_ref, b1_ref, u2_ref):
    dv = dv_ref[...]
    z = dv * (s0_ref[...] + s1_ref[...] + u1_ref[...]) + b1_ref[...][0][None, :]
    u2_ref[...] = dv * jnp.maximum(z, 0.0)


def _tc_c_body(s0_ref, s1_ref, u2_ref, dv_ref, w2_ref, b2_ref, o_ref):
    agg = dv_ref[...] * (s0_ref[...] + s1_ref[...] + u2_ref[...])
    y = jnp.dot(agg, w2_ref[...], preferred_element_type=jnp.float32)
    y = y + b2_ref[...][0][None, :]
    m = jnp.max(y, axis=1, keepdims=True)
    lse = jnp.log(jnp.sum(jnp.exp(y - m), axis=1, keepdims=True)) + m
    o_ref[...] = y - lse


def kernel(x, edge_index, W1, b1, W2, b2):
    n, din = x.shape
    e = edge_index.shape[1]
    h = W1.shape[1]
    dout = W2.shape[1]
    assert h == _LANES

    n_pad = -(-n // (_NS * _CH)) * (_NS * _CH)
    cpt = -(-(-(-e // (_NT * _CH))) // 16) * 16
    e_pad = _NT * cpt * _CH

    xp = jnp.pad(x, ((0, n_pad - n), (0, 0)))
    trash = n + jnp.arange(e_pad - e, dtype=jnp.int32) % (n_pad - n)
    src = jnp.concatenate([edge_index[0], trash]).reshape(_NT * cpt, _CH)
    dst = jnp.concatenate([edge_index[1], trash]).reshape(_NT * cpt, _CH)
    fill = jnp.stack([jnp.zeros((_CH, _LANES), jnp.float32),
                      jnp.ones((_CH, _LANES), jnp.float32)])

    br = 1024
    grid = (n_pad // br,)
    row_spec = pl.BlockSpec((br, _LANES), lambda i: (i, 0))

    degp = _sc_pass(None, dst, None, fill, n_pad, cpt, gather=False)

    hm = pl.pallas_call(
        _tc_mm_body,
        grid=grid,
        in_specs=[pl.BlockSpec((br, din), lambda i: (i, 0)),
                  pl.BlockSpec((din, h), lambda i: (0, 0))],
        out_specs=row_spec,
        out_shape=jax.ShapeDtypeStruct((n_pad, h), jnp.float32),
    )(xp, W1)

    u1, dv = pl.pallas_call(
        _tc_a_body,
        grid=grid,
        in_specs=[row_spec, row_spec, row_spec],
        out_specs=[row_spec, row_spec],
        out_shape=[jax.ShapeDtypeStruct((n_pad, h), jnp.float32),
                   jax.ShapeDtypeStruct((n_pad, h), jnp.float32)],
    )(hm, degp[0], degp[1])

    s1p = _sc_pass(src, dst, u1, fill, n_pad, cpt, gather=True)

    u2 = pl.pallas_call(
        _tc_b_body,
        grid=grid,
        in_specs=[row_spec, row_spec, row_spec, row_spec,
                  pl.BlockSpec((8, h), lambda i: (0, 0))],
        out_specs=row_spec,
        out_shape=jax.ShapeDtypeStruct((n_pad, h), jnp.float32),
    )(s1p[0], s1p[1], u1, dv, jnp.broadcast_to(b1, (8, h)))

    s2p = _sc_pass(src, dst, u2, fill, n_pad, cpt, gather=True)

    out = pl.pallas_call(
        _tc_c_body,
        grid=grid,
        in_specs=[row_spec, row_spec, row_spec, row_spec,
                  pl.BlockSpec((h, dout), lambda i: (0, 0)),
                  pl.BlockSpec((8, dout), lambda i: (0, 0))],
        out_specs=pl.BlockSpec((br, dout), lambda i: (i, 0)),
        out_shape=jax.ShapeDtypeStruct((n_pad, dout), jnp.float32),
    )(s2p[0], s2p[1], u2, dv, W2, jnp.broadcast_to(b2, (8, dout)))

    return out[:n]


# speedup vs baseline: 1.1216x; 1.0215x over previous
import functools

import jax
import jax.numpy as jnp
from jax import lax
from jax.experimental import pallas as pl
from jax.experimental.pallas import tpu as pltpu
from jax.experimental.pallas import tpu_sc as plsc

_NC = 2
_NS = 16
_NT = _NC * _NS
_CH = 128
_LANES = 16


def _sc_mesh():
    return plsc.VectorSubcoreMesh(core_axis_name="c", subcore_axis_name="s")


def _sc_pass(src, dst, u, fill, n_pad, cpt, gather):
    rpt = n_pad // _NS
    nb = 4
    assert cpt % nb == 0
    scratch = [
        pltpu.VMEM((cpt, _CH), jnp.int32),
        pltpu.VMEM((nb, _CH, _LANES), jnp.float32),
        pltpu.VMEM_SHARED((n_pad, _LANES), jnp.float32),
        pltpu.SemaphoreType.DMA((nb,)),
        pltpu.SemaphoreType.DMA((nb,)),
    ]
    if gather:
        scratch.insert(0, pltpu.VMEM((cpt, _CH), jnp.int32))
        scratch.insert(3, pltpu.VMEM_SHARED((n_pad, _LANES), jnp.float32))

    def body(*refs):
        if gather:
            (src_hbm, dst_hbm, u_hbm, fill_hbm, out_hbm,
             src_v, dst_v, rows_v, u_sh, acc_sh, gsem, ssem) = refs
        else:
            (dst_hbm, fill_hbm, out_hbm, dst_v, rows_v, acc_sh,
             gsem, ssem) = refs
        c = lax.axis_index("c")
        s = lax.axis_index("s")
        wid = c * _NS + s

        pltpu.sync_copy(fill_hbm.at[0], rows_v.at[0])

        @pl.loop(0, rpt // _CH)
        def _(i):
            pltpu.sync_copy(rows_v.at[0],
                            acc_sh.at[pl.ds(s * rpt + i * _CH, _CH)])

        pltpu.sync_copy(dst_hbm.at[pl.ds(wid * cpt, cpt)], dst_v)
        if gather:
            pltpu.sync_copy(src_hbm.at[pl.ds(wid * cpt, cpt)], src_v)
            pltpu.sync_copy(u_hbm.at[pl.ds(s * rpt, rpt)],
                            u_sh.at[pl.ds(s * rpt, rpt)])
        else:
            for b in range(nb):
                pltpu.sync_copy(fill_hbm.at[1], rows_v.at[b])
        plsc.subcore_barrier()

        if gather:
            for b in range(nb):
                pltpu.async_copy(u_sh.at[src_v.at[b]], rows_v.at[b],
                                 gsem.at[b])

            @pl.loop(0, cpt // nb)
            def _(t):
                j0 = t * nb
                for b in range(nb):
                    pltpu.make_async_copy(u_sh.at[src_v.at[j0 + b]],
                                          rows_v.at[b], gsem.at[b]).wait()
                    pltpu.async_copy(rows_v.at[b],
                                     acc_sh.at[dst_v.at[j0 + b]],
                                     ssem.at[b], add=True)
                for b in range(nb):
                    jn = jnp.minimum(j0 + b + nb, cpt - 1)
                    pltpu.make_async_copy(rows_v.at[b],
                                          acc_sh.at[dst_v.at[j0 + b]],
                                          ssem.at[b]).wait()
                    pltpu.async_copy(u_sh.at[src_v.at[jn]], rows_v.at[b],
                                     gsem.at[b])

            for b in range(nb):
                pltpu.make_async_copy(u_sh.at[src_v.at[0]], rows_v.at[b],
                                      gsem.at[b]).wait()
        else:
            for b in range(nb):
                pltpu.async_copy(rows_v.at[b], acc_sh.at[dst_v.at[b]],
                                 ssem.at[b], add=True)

            @pl.loop(1, cpt // nb)
            def _(t):
                for b in range(nb):
                    pltpu.make_async_copy(rows_v.at[b],
                                          acc_sh.at[dst_v.at[b]],
                                          ssem.at[b]).wait()
                    pltpu.async_copy(rows_v.at[b],
                                     acc_sh.at[dst_v.at[t * nb + b]],
                                     ssem.at[b], add=True)

            for b in range(nb):
                pltpu.make_async_copy(rows_v.at[b], acc_sh.at[dst_v.at[0]],
                                      ssem.at[b]).wait()

        plsc.subcore_barrier()
        pltpu.sync_copy(acc_sh.at[pl.ds(s * rpt, rpt)],
                        out_hbm.at[c, pl.ds(s * rpt, rpt)])

    k = pl.kernel(
        body,
        out_type=jax.ShapeDtypeStruct((_NC, n_pad, _LANES), jnp.float32),
        mesh=_sc_mesh(),
        scratch_types=scratch,
        compiler_params=pltpu.CompilerParams(use_tc_tiling_on_sc=False),
    )
    if gather:
        return k(src, dst, u, fill)
    return k(dst, fill)


def _tc_a_body(x_ref, w_ref, d0_ref, d1_ref, u_ref, dv_ref):
    deg = d0_ref[...] + d1_ref[...] + 1.0
    dinv = lax.rsqrt(deg)
    h = jnp.dot(x_ref[...], w_ref[...], preferred_element_type=jnp.float32)
    u_ref[...] = dinv * h
    dv_ref[...] = dinv


def _tc_b_body(s0_ref, s1_ref, u1_ref, dv_ref, b1_ref, u2_ref):
    dv = dv_ref[...]
    z = dv * (s0_ref[...] + s1_ref[...] + u1_ref[...]) + b1_ref[...][0][None, :]
    u2_ref[...] = dv * jnp.maximum(z, 0.0)


def _tc_c_body(s0_ref, s1_ref, u2_ref, dv_ref, w2_ref, b2_ref, o_ref):
    agg = dv_ref[...] * (s0_ref[...] + s1_ref[...] + u2_ref[...])
    y = jnp.dot(agg, w2_ref[...], preferred_element_type=jnp.float32)
    y = y + b2_ref[...][0][None, :]
    m = jnp.max(y, axis=1, keepdims=True)
    lse = jnp.log(jnp.sum(jnp.exp(y - m), axis=1, keepdims=True)) + m
    o_ref[...] = y - lse


def kernel(x, edge_index, W1, b1, W2, b2):
    n, din = x.shape
    e = edge_index.shape[1]
    h = W1.shape[1]
    dout = W2.shape[1]
    assert h == _LANES

    n_pad = -(-n // (_NS * _CH)) * (_NS * _CH)
    cpt = -(-(-(-e // (_NT * _CH))) // 16) * 16
    e_pad = _NT * cpt * _CH

    trash = n + jnp.arange(e_pad - e, dtype=jnp.int32) % (n_pad - n)
    src = jnp.concatenate([edge_index[0], trash]).reshape(_NT * cpt, _CH)
    dst = jnp.concatenate([edge_index[1], trash]).reshape(_NT * cpt, _CH)
    fill = jnp.stack([jnp.zeros((_CH, _LANES), jnp.float32),
                      jnp.ones((_CH, _LANES), jnp.float32)])

    br = 1024
    grid = (n_pad // br,)
    row_spec = pl.BlockSpec((br, _LANES), lambda i: (i, 0))

    degp = _sc_pass(None, dst, None, fill, n_pad, cpt, gather=False)

    u1, dv = pl.pallas_call(
        _tc_a_body,
        grid=grid,
        in_specs=[pl.BlockSpec((br, din), lambda i: (i, 0)),
                  pl.BlockSpec((din, h), lambda i: (0, 0)),
                  row_spec, row_spec],
        out_specs=[row_spec, row_spec],
        out_shape=[jax.ShapeDtypeStruct((n_pad, h), jnp.float32),
                   jax.ShapeDtypeStruct((n_pad, h), jnp.float32)],
    )(x, W1, degp[0], degp[1])

    s1p = _sc_pass(src, dst, u1, fill, n_pad, cpt, gather=True)

    u2 = pl.pallas_call(
        _tc_b_body,
        grid=grid,
        in_specs=[row_spec, row_spec, row_spec, row_spec,
                  pl.BlockSpec((8, h), lambda i: (0, 0))],
        out_specs=row_spec,
        out_shape=jax.ShapeDtypeStruct((n_pad, h), jnp.float32),
    )(s1p[0], s1p[1], u1, dv, jnp.broadcast_to(b1, (8, h)))

    s2p = _sc_pass(src, dst, u2, fill, n_pad, cpt, gather=True)

    out = pl.pallas_call(
        _tc_c_body,
        grid=grid,
        in_specs=[row_spec, row_spec, row_spec, row_spec,
                  pl.BlockSpec((h, dout), lambda i: (0, 0)),
                  pl.BlockSpec((8, dout), lambda i: (0, 0))],
        out_specs=pl.BlockSpec((br, dout), lambda i: (i, 0)),
        out_shape=jax.ShapeDtypeStruct((n_pad, dout), jnp.float32),
    )(s2p[0], s2p[1], u2, dv, W2, jnp.broadcast_to(b2, (8, dout)))

    return out[:n]


# speedup vs baseline: 1.1420x; 1.0182x over previous
import functools

import jax
import jax.numpy as jnp
from jax import lax
from jax.experimental import pallas as pl
from jax.experimental.pallas import tpu as pltpu
from jax.experimental.pallas import tpu_sc as plsc

_NC = 2
_NS = 16
_NT = _NC * _NS
_CH = 128
_LANES = 16


def _sc_mesh():
    return plsc.VectorSubcoreMesh(core_axis_name="c", subcore_axis_name="s")


def _sc_pass(src, dst, u, fill, n_pad, cpt, gather):
    rpt = n_pad // _NS
    nb = 4
    assert cpt % nb == 0
    scratch = [
        pltpu.VMEM((cpt, _CH), jnp.int32),
        pltpu.VMEM((nb, _CH, _LANES), jnp.float32),
        pltpu.VMEM_SHARED((n_pad, _LANES), jnp.float32),
        pltpu.SemaphoreType.DMA((nb,)),
        pltpu.SemaphoreType.DMA((nb,)),
    ]
    if gather:
        scratch.insert(0, pltpu.VMEM((cpt, _CH), jnp.int32))
        scratch.insert(3, pltpu.VMEM_SHARED((n_pad, _LANES), jnp.float32))
        scratch.append(pltpu.SemaphoreType.DMA((3,)))

    def body(*refs):
        if gather:
            (src_hbm, dst_hbm, u_hbm, fill_hbm, out_hbm,
             src_v, dst_v, rows_v, u_sh, acc_sh, gsem, ssem, psem) = refs
        else:
            (dst_hbm, fill_hbm, out_hbm, dst_v, rows_v, acc_sh,
             gsem, ssem) = refs
        c = lax.axis_index("c")
        s = lax.axis_index("s")
        wid = c * _NS + s

        if gather:
            idx_d = pltpu.async_copy(dst_hbm.at[pl.ds(wid * cpt, cpt)],
                                     dst_v, psem.at[0])
            idx_s = pltpu.async_copy(src_hbm.at[pl.ds(wid * cpt, cpt)],
                                     src_v, psem.at[1])
            stage_u = pltpu.async_copy(u_hbm.at[pl.ds(s * rpt, rpt)],
                                       u_sh.at[pl.ds(s * rpt, rpt)],
                                       psem.at[2])
        else:
            pltpu.sync_copy(dst_hbm.at[pl.ds(wid * cpt, cpt)], dst_v)

        pltpu.sync_copy(fill_hbm.at[0], rows_v.at[0])

        @pl.loop(0, rpt // _CH)
        def _(i):
            pltpu.sync_copy(rows_v.at[0],
                            acc_sh.at[pl.ds(s * rpt + i * _CH, _CH)])

        if gather:
            idx_d.wait()
            idx_s.wait()
            stage_u.wait()
        else:
            for b in range(nb):
                pltpu.sync_copy(fill_hbm.at[1], rows_v.at[b])
        plsc.subcore_barrier()

        if gather:
            for b in range(nb):
                pltpu.async_copy(u_sh.at[src_v.at[b]], rows_v.at[b],
                                 gsem.at[b])

            @pl.loop(0, cpt // nb)
            def _(t):
                j0 = t * nb
                for b in range(nb):
                    pltpu.make_async_copy(u_sh.at[src_v.at[j0 + b]],
                                          rows_v.at[b], gsem.at[b]).wait()
                    pltpu.async_copy(rows_v.at[b],
                                     acc_sh.at[dst_v.at[j0 + b]],
                                     ssem.at[b], add=True)
                for b in range(nb):
                    jn = jnp.minimum(j0 + b + nb, cpt - 1)
                    pltpu.make_async_copy(rows_v.at[b],
                                          acc_sh.at[dst_v.at[j0 + b]],
                                          ssem.at[b]).wait()
                    pltpu.async_copy(u_sh.at[src_v.at[jn]], rows_v.at[b],
                                     gsem.at[b])

            for b in range(nb):
                pltpu.make_async_copy(u_sh.at[src_v.at[0]], rows_v.at[b],
                                      gsem.at[b]).wait()
        else:
            for b in range(nb):
                pltpu.async_copy(rows_v.at[b], acc_sh.at[dst_v.at[b]],
                                 ssem.at[b], add=True)

            @pl.loop(1, cpt // nb)
            def _(t):
                for b in range(nb):
                    pltpu.make_async_copy(rows_v.at[b],
                                          acc_sh.at[dst_v.at[b]],
                                          ssem.at[b]).wait()
                    pltpu.async_copy(rows_v.at[b],
                                     acc_sh.at[dst_v.at[t * nb + b]],
                                     ssem.at[b], add=True)

            for b in range(nb):
                pltpu.make_async_copy(rows_v.at[b], acc_sh.at[dst_v.at[0]],
                                      ssem.at[b]).wait()

        plsc.subcore_barrier()
        pltpu.sync_copy(acc_sh.at[pl.ds(s * rpt, rpt)],
                        out_hbm.at[c, pl.ds(s * rpt, rpt)])

    k = pl.kernel(
        body,
        out_type=jax.ShapeDtypeStruct((_NC, n_pad, _LANES), jnp.float32),
        mesh=_sc_mesh(),
        scratch_types=scratch,
        compiler_params=pltpu.CompilerParams(use_tc_tiling_on_sc=False),
    )
    if gather:
        return k(src, dst, u, fill)
    return k(dst, fill)


def _tc_a_body(x_ref, w_ref, d0_ref, d1_ref, u_ref, dv_ref):
    deg = d0_ref[...] + d1_ref[...] + 1.0
    dinv = lax.rsqrt(deg)
    h = jnp.dot(x_ref[...], w_ref[...], preferred_element_type=jnp.float32)
    u_ref[...] = dinv * h
    dv_ref[...] = dinv


def _tc_b_body(s0_ref, s1_ref, u1_ref, dv_ref, b1_ref, u2_ref):
    dv = dv_ref[...]
    z = dv * (s0_ref[...] + s1_ref[...] + u1_ref[...]) + b1_ref[...][0][None, :]
    u2_ref[...] = dv * jnp.maximum(z, 0.0)


def _tc_c_body(s0_ref, s1_ref, u2_ref, dv_ref, w2_ref, b2_ref, o_ref):
    agg = dv_ref[...] * (s0_ref[...] + s1_ref[...] + u2_ref[...])
    y = jnp.dot(agg, w2_ref[...], preferred_element_type=jnp.float32)
    y = y + b2_ref[...][0][None, :]
    m = jnp.max(y, axis=1, keepdims=True)
    lse = jnp.log(jnp.sum(jnp.exp(y - m), axis=1, keepdims=True)) + m
    o_ref[...] = y - lse


def kernel(x, edge_index, W1, b1, W2, b2):
    n, din = x.shape
    e = edge_index.shape[1]
    h = W1.shape[1]
    dout = W2.shape[1]
    assert h == _LANES

    n_pad = -(-n // (_NS * _CH)) * (_NS * _CH)
    cpt = -(-(-(-e // (_NT * _CH))) // 16) * 16
    e_pad = _NT * cpt * _CH

    trash = n + jnp.arange(e_pad - e, dtype=jnp.int32) % (n_pad - n)
    src = jnp.concatenate([edge_index[0], trash]).reshape(_NT * cpt, _CH)
    dst = jnp.concatenate([edge_index[1], trash]).reshape(_NT * cpt, _CH)
    fill = jnp.stack([jnp.zeros((_CH, _LANES), jnp.float32),
                      jnp.ones((_CH, _LANES), jnp.float32)])

    br = 1024
    grid = (n_pad // br,)
    row_spec = pl.BlockSpec((br, _LANES), lambda i: (i, 0))

    degp = _sc_pass(None, dst, None, fill, n_pad, cpt, gather=False)

    u1, dv = pl.pallas_call(
        _tc_a_body,
        grid=grid,
        in_specs=[pl.BlockSpec((br, din), lambda i: (i, 0)),
                  pl.BlockSpec((din, h), lambda i: (0, 0)),
                  row_spec, row_spec],
        out_specs=[row_spec, row_spec],
        out_shape=[jax.ShapeDtypeStruct((n_pad, h), jnp.float32),
                   jax.ShapeDtypeStruct((n_pad, h), jnp.float32)],
    )(x, W1, degp[0], degp[1])

    s1p = _sc_pass(src, dst, u1, fill, n_pad, cpt, gather=True)

    u2 = pl.pallas_call(
        _tc_b_body,
        grid=grid,
        in_specs=[row_spec, row_spec, row_spec, row_spec,
                  pl.BlockSpec((8, h), lambda i: (0, 0))],
        out_specs=row_spec,
        out_shape=jax.ShapeDtypeStruct((n_pad, h), jnp.float32),
    )(s1p[0], s1p[1], u1, dv, jnp.broadcast_to(b1, (8, h)))

    s2p = _sc_pass(src, dst, u2, fill, n_pad, cpt, gather=True)

    out = pl.pallas_call(
        _tc_c_body,
        grid=grid,
        in_specs=[row_spec, row_spec, row_spec, row_spec,
                  pl.BlockSpec((h, dout), lambda i: (0, 0)),
                  pl.BlockSpec((8, dout), lambda i: (0, 0))],
        out_specs=pl.BlockSpec((br, dout), lambda i: (i, 0)),
        out_shape=jax.ShapeDtypeStruct((n_pad, dout), jnp.float32),
    )(s2p[0], s2p[1], u2, dv, W2, jnp.broadcast_to(b2, (8, dout)))

    return out[:n]


# speedup vs baseline: 1.1461x; 1.0036x over previous
import functools

import jax
import jax.numpy as jnp
from jax import lax
from jax.experimental import pallas as pl
from jax.experimental.pallas import tpu as pltpu
from jax.experimental.pallas import tpu_sc as plsc

_NC = 2
_NS = 16
_NT = _NC * _NS
_CH = 128
_LANES = 16


def _sc_mesh():
    return plsc.VectorSubcoreMesh(core_axis_name="c", subcore_axis_name="s")


def _sc_pass(src, dst, u, fill, n_pad, cpt, gather):
    rpt = n_pad // _NS
    nb = 4
    assert cpt % nb == 0
    scratch = [
        pltpu.VMEM((cpt, _CH), jnp.int32),
        pltpu.VMEM((nb, _CH, _LANES), jnp.float32),
        pltpu.VMEM_SHARED((n_pad, _LANES), jnp.float32),
        pltpu.SemaphoreType.DMA((nb,)),
        pltpu.SemaphoreType.DMA((nb,)),
    ]
    if gather:
        scratch.insert(0, pltpu.VMEM((cpt, _CH), jnp.int32))
        scratch.insert(3, pltpu.VMEM_SHARED((n_pad, _LANES), jnp.float32))
    scratch.append(pltpu.SemaphoreType.DMA((3,)))

    def body(*refs):
        if gather:
            (src_hbm, dst_hbm, u_hbm, fill_hbm, out_hbm,
             src_v, dst_v, rows_v, u_sh, acc_sh, gsem, ssem, psem) = refs
        else:
            (dst_hbm, fill_hbm, out_hbm, dst_v, rows_v, acc_sh,
             gsem, ssem, psem) = refs
        c = lax.axis_index("c")
        s = lax.axis_index("s")
        wid = c * _NS + s

        if gather:
            idx_d = pltpu.async_copy(dst_hbm.at[pl.ds(wid * cpt, cpt)],
                                     dst_v, psem.at[0])
            idx_s = pltpu.async_copy(src_hbm.at[pl.ds(wid * cpt, cpt)],
                                     src_v, psem.at[1])
            stage_u = pltpu.async_copy(u_hbm.at[pl.ds(s * rpt, rpt)],
                                       u_sh.at[pl.ds(s * rpt, rpt)],
                                       psem.at[2])
        else:
            idx_d = pltpu.async_copy(dst_hbm.at[pl.ds(wid * cpt, cpt)],
                                     dst_v, psem.at[0])

        pltpu.sync_copy(fill_hbm.at[0], rows_v.at[0])

        @pl.loop(0, rpt // _CH)
        def _(i):
            pltpu.sync_copy(rows_v.at[0],
                            acc_sh.at[pl.ds(s * rpt + i * _CH, _CH)])

        if gather:
            idx_d.wait()
            idx_s.wait()
            stage_u.wait()
        else:
            for b in range(nb):
                pltpu.sync_copy(fill_hbm.at[1], rows_v.at[b])
            idx_d.wait()
        plsc.subcore_barrier()

        if gather:
            for b in range(nb):
                pltpu.async_copy(u_sh.at[src_v.at[b]], rows_v.at[b],
                                 gsem.at[b])

            @pl.loop(0, cpt // nb)
            def _(t):
                j0 = t * nb
                for b in range(nb):
                    pltpu.make_async_copy(u_sh.at[src_v.at[j0 + b]],
                                          rows_v.at[b], gsem.at[b]).wait()
                    pltpu.async_copy(rows_v.at[b],
                                     acc_sh.at[dst_v.at[j0 + b]],
                                     ssem.at[b], add=True)
                for b in range(nb):
                    jn = jnp.minimum(j0 + b + nb, cpt - 1)
                    pltpu.make_async_copy(rows_v.at[b],
                                          acc_sh.at[dst_v.at[j0 + b]],
                                          ssem.at[b]).wait()
                    pltpu.async_copy(u_sh.at[src_v.at[jn]], rows_v.at[b],
                                     gsem.at[b])

            for b in range(nb):
                pltpu.make_async_copy(u_sh.at[src_v.at[0]], rows_v.at[b],
                                      gsem.at[b]).wait()
        else:
            for b in range(nb):
                pltpu.async_copy(rows_v.at[b], acc_sh.at[dst_v.at[b]],
                                 ssem.at[b], add=True)

            @pl.loop(1, cpt // nb)
            def _(t):
                for b in range(nb):
                    pltpu.make_async_copy(rows_v.at[b],
                                          acc_sh.at[dst_v.at[b]],
                                          ssem.at[b]).wait()
                    pltpu.async_copy(rows_v.at[b],
                                     acc_sh.at[dst_v.at[t * nb + b]],
                                     ssem.at[b], add=True)

            for b in range(nb):
                pltpu.make_async_copy(rows_v.at[b], acc_sh.at[dst_v.at[0]],
                                      ssem.at[b]).wait()

        plsc.subcore_barrier()
        pltpu.sync_copy(acc_sh.at[pl.ds(s * rpt, rpt)],
                        out_hbm.at[c, pl.ds(s * rpt, rpt)])

    k = pl.kernel(
        body,
        out_type=jax.ShapeDtypeStruct((_NC, n_pad, _LANES), jnp.float32),
        mesh=_sc_mesh(),
        scratch_types=scratch,
        compiler_params=pltpu.CompilerParams(use_tc_tiling_on_sc=False),
    )
    if gather:
        return k(src, dst, u, fill)
    return k(dst, fill)


def _tc_a_body(x_ref, w_ref, d0_ref, d1_ref, u_ref, dv_ref):
    deg = d0_ref[...] + d1_ref[...] + 1.0
    dinv = lax.rsqrt(deg)
    h = jnp.dot(x_ref[...], w_ref[...], preferred_element_type=jnp.float32)
    u_ref[...] = dinv * h
    dv_ref[...] = dinv


def _tc_b_body(s0_ref, s1_ref, u1_ref, dv_ref, b1_ref, u2_ref):
    dv = dv_ref[...]
    z = dv * (s0_ref[...] + s1_ref[...] + u1_ref[...]) + b1_ref[...][0][None, :]
    u2_ref[...] = dv * jnp.maximum(z, 0.0)


def _tc_c_body(s0_ref, s1_ref, u2_ref, dv_ref, w2_ref, b2_ref, o_ref):
    agg = dv_ref[...] * (s0_ref[...] + s1_ref[...] + u2_ref[...])
    y = jnp.dot(agg, w2_ref[...], preferred_element_type=jnp.float32)
    y = y + b2_ref[...][0][None, :]
    m = jnp.max(y, axis=1, keepdims=True)
    lse = jnp.log(jnp.sum(jnp.exp(y - m), axis=1, keepdims=True)) + m
    o_ref[...] = y - lse


def kernel(x, edge_index, W1, b1, W2, b2):
    n, din = x.shape
    e = edge_index.shape[1]
    h = W1.shape[1]
    dout = W2.shape[1]
    assert h == _LANES

    n_pad = -(-n // (_NS * _CH)) * (_NS * _CH)
    cpt = -(-(-(-e // (_NT * _CH))) // 16) * 16
    e_pad = _NT * cpt * _CH

    trash = n + jnp.arange(e_pad - e, dtype=jnp.int32) % (n_pad - n)
    src = jnp.concatenate([edge_index[0], trash]).reshape(_NT * cpt, _CH)
    dst = jnp.concatenate([edge_index[1], trash]).reshape(_NT * cpt, _CH)
    fill = jnp.stack([jnp.zeros((_CH, _LANES), jnp.float32),
                      jnp.ones((_CH, _LANES), jnp.float32)])

    br = 1024
    grid = (n_pad // br,)
    row_spec = pl.BlockSpec((br, _LANES), lambda i: (i, 0))

    degp = _sc_pass(None, dst, None, fill, n_pad, cpt, gather=False)

    u1, dv = pl.pallas_call(
        _tc_a_body,
        grid=grid,
        in_specs=[pl.BlockSpec((br, din), lambda i: (i, 0)),
                  pl.BlockSpec((din, h), lambda i: (0, 0)),
                  row_spec, row_spec],
        out_specs=[row_spec, row_spec],
        out_shape=[jax.ShapeDtypeStruct((n_pad, h), jnp.float32),
                   jax.ShapeDtypeStruct((n_pad, h), jnp.float32)],
    )(x, W1, degp[0], degp[1])

    s1p = _sc_pass(src, dst, u1, fill, n_pad, cpt, gather=True)

    u2 = pl.pallas_call(
        _tc_b_body,
        grid=grid,
        in_specs=[row_spec, row_spec, row_spec, row_spec,
                  pl.BlockSpec((8, h), lambda i: (0, 0))],
        out_specs=row_spec,
        out_shape=jax.ShapeDtypeStruct((n_pad, h), jnp.float32),
    )(s1p[0], s1p[1], u1, dv, jnp.broadcast_to(b1, (8, h)))

    s2p = _sc_pass(src, dst, u2, fill, n_pad, cpt, gather=True)

    out = pl.pallas_call(
        _tc_c_body,
        grid=grid,
        in_specs=[row_spec, row_spec, row_spec, row_spec,
                  pl.BlockSpec((h, dout), lambda i: (0, 0)),
                  pl.BlockSpec((8, dout), lambda i: (0, 0))],
        out_specs=pl.BlockSpec((br, dout), lambda i: (i, 0)),
        out_shape=jax.ShapeDtypeStruct((n_pad, dout), jnp.float32),
    )(s2p[0], s2p[1], u2, dv, W2, jnp.broadcast_to(b2, (8, dout)))

    return out[:n]
